# baseline (speedup 1.0000x reference)
import functools

import jax
import jax.numpy as jnp
import numpy as np
from jax import lax
from jax.experimental import pallas as pl
from jax.experimental.pallas import tpu as pltpu
from jax.experimental.pallas import tpu_sc as plsc

_ANCHORS = np.array(
    [[1.3221, 1.73145], [3.19275, 4.00944], [5.05587, 8.09892],
     [9.47112, 4.84053], [11.2364, 10.0071]], dtype=np.float32)
_A = 5
_C = 20
_S = 13
_YX = _S * _S
_CH = 5 + _C


def _sel_np():
    s = np.zeros((_A * _CH, 5 * _A), dtype=np.float32)
    for f in range(5):
        for a in range(_A):
            s[a * _CH + f, f * _A + a] = 1.0
    return s


def _anc_planes_np():
    ys, xs = np.meshgrid(np.arange(_S, dtype=np.float32),
                         np.arange(_S, dtype=np.float32), indexing='ij')
    cx = (xs + 0.5).reshape(_YX)
    cy = (ys + 0.5).reshape(_YX)
    out = np.zeros((4 * _A, _YX), dtype=np.float32)
    for a in range(_A):
        out[0 * _A + a] = cx
        out[1 * _A + a] = cy
        out[2 * _A + a] = _ANCHORS[a, 0]
        out[3 * _A + a] = _ANCHORS[a, 1]
    return out


_SEL = _sel_np()
_ANC_PLANES = _anc_planes_np()

_G = 32

_NOBJ = 1024
_CHUNK = 16
_NW = 32
_PER_W = _NOBJ // (_CHUNK * _NW)


def _sc_perm(x, idx):
    return lax.gather(
        x, idx.reshape(16, 1),
        lax.GatherDimensionNumbers(offset_dims=(), collapsed_slice_dims=(0,),
                                   start_index_map=(0,)),
        (1,), mode=lax.GatherScatterMode.PROMISE_IN_BOUNDS)


def _sc_assign_kernel(packed, meta, in_v, out_v):
    cid = lax.axis_index("c")
    sid = lax.axis_index("s")
    wid = sid * 2 + cid
    lane = lax.iota(jnp.int32, 16)
    pos = lane & 7
    for k in range(_PER_W):
        c = wid * _PER_W + k
        pltpu.sync_copy(packed.at[pl.ds(c * 80, 80)], in_v)
        x1 = in_v[pl.ds(0, 16)] * float(_S)
        y1 = in_v[pl.ds(16, 16)] * float(_S)
        x2 = in_v[pl.ds(32, 16)] * float(_S)
        y2 = in_v[pl.ds(48, 16)] * float(_S)
        area_g = (x2 - x1) * (y2 - y1)
        bx = (x1 + x2) * 0.5
        by = (y1 + y2) * 0.5
        bw = x2 - x1
        bh = y2 - y1
        cxf = bx.astype(jnp.int32).astype(jnp.float32)
        cyf = by.astype(jnp.int32).astype(jnp.float32)
        tx = bx - cxf
        ty = by - cyf

        best = jnp.full((16,), -1.0, jnp.float32)
        jb = jnp.zeros((16,), jnp.float32)
        awj = jnp.ones((16,), jnp.float32)
        ahj = jnp.ones((16,), jnp.float32)
        for a in range(_A):
            aw = float(_ANCHORS[a, 0])
            ah = float(_ANCHORS[a, 1])
            iwx = jnp.maximum(
                jnp.minimum(cxf + 0.5 + aw * 0.5, x2)
                - jnp.maximum(cxf + 0.5 - aw * 0.5, x1), 0.0)
            iwy = jnp.maximum(
                jnp.minimum(cyf + 0.5 + ah * 0.5, y2)
                - jnp.maximum(cyf + 0.5 - ah * 0.5, y1), 0.0)
            ai = iwx * iwy
            aiou = ai / (aw * ah + area_g - ai)
            take = aiou > best
            best = jnp.maximum(best, aiou)
            jb = jnp.where(take, float(a), jb)
            awj = jnp.where(take, aw, awj)
            ahj = jnp.where(take, ah, ahj)

        yx = cyf * float(_S) + cxf
        key = yx * float(_A) + jb
        clsf = in_v[pl.ds(64, 16)] - 1.0

        later_eq = jnp.zeros((16,), jnp.float32)
        later_pair = jnp.zeros((16,), jnp.float32)
        for dd in range(1, 8):
            valid = (pos + dd) < 8
            perm = lane + jnp.where(valid, dd, 0)
            kg = _sc_perm(key, perm)
            cg = _sc_perm(clsf, perm)
            hit = jnp.where(valid & (kg == key), 1.0, 0.0)
            later_eq = jnp.maximum(later_eq, hit)
            later_pair = jnp.maximum(
                later_pair, jnp.where(cg == clsf, hit, 0.0))
        winner = 1.0 - later_eq
        uniq = 1.0 - later_pair

        fields = [yx, jb, tx, ty, bw / awj, bh / ahj, clsf, winner, uniq]
        zero = jnp.zeros((16,), jnp.float32)
        for f in range(16):
            out_v[pl.ds(f * 16, 16)] = fields[f] if f < 9 else zero
        pltpu.sync_copy(out_v, meta.at[pl.ds(c * 256, 256)])


@functools.partial(
    pl.kernel,
    mesh=plsc.VectorSubcoreMesh(core_axis_name="c", subcore_axis_name="s"),
    out_type=jax.ShapeDtypeStruct((_NOBJ * 16,), jnp.float32),
    scratch_types=[
        pltpu.VMEM((80,), jnp.float32),
        pltpu.VMEM((256,), jnp.float32),
    ],
)
def _sc_assign(packed, meta, in_v, out_v):
    _sc_assign_kernel(packed, meta, in_v, out_v)


def _tc_kernel(gt_smem, meta_ref, pred_ref, anc_ref, sel_ref, out_ref):
    b = pl.program_id(0)

    @pl.when(b == 0)
    def _init():
        out_ref[...] = jnp.zeros_like(out_ref)

    acc = jnp.zeros((8, 128), jnp.float32)
    for g in range(_G):
        acc = acc + _one_image(pred_ref, g, b * _G + g, gt_smem,
                               meta_ref[g // 2], g % 2,
                               anc_ref[...], sel_ref[...])
    out_ref[...] += acc


def _one_image(pred_ref, g, bb, gt_smem, meta, h, anc, sel):
    pt = pred_ref[g]

    t = lax.dot_general(sel, pt, (((0,), (1,)), ((), ())))
    pxy = jax.nn.sigmoid(t[0:10])
    pwh = jnp.exp(t[10:20])
    pconf = jax.nn.sigmoid(t[20:25])

    cpx = anc[0:5] + pxy[0:5]
    cpy = anc[5:10] + pxy[5:10]
    cpw = anc[10:15] * pwh[0:5]
    cph = anc[15:20] * pwh[5:10]
    px1 = cpx - cpw * 0.5
    py1 = cpy - cph * 0.5
    px2 = cpx + cpw * 0.5
    py2 = cpy + cph * 0.5
    area_p = cpw * cph

    gt_conf = None
    for n in range(8):
        sx1 = gt_smem[bb, n, 0] * float(_S)
        sy1 = gt_smem[bb, n, 1] * float(_S)
        sx2 = gt_smem[bb, n, 2] * float(_S)
        sy2 = gt_smem[bb, n, 3] * float(_S)
        sarea = (sx2 - sx1) * (sy2 - sy1)
        tlx = jnp.maximum(px1, sx1)
        tly = jnp.maximum(py1, sy1)
        brx = jnp.minimum(px2, sx2)
        bry = jnp.minimum(py2, sy2)
        wx = jnp.maximum(brx - tlx, 0.0)
        wy = jnp.maximum(bry - tly, 0.0)
        inter = wx * wy
        iou = inter / (area_p + (sarea - inter))
        gt_conf = iou if gt_conf is None else jnp.maximum(gt_conf, iou)

    d = gt_conf - pconf
    dense_sum = jnp.sum(d * d)

    lo = h * 8
    yx_r = meta[0:1, lo:lo + 8]
    jb_r = meta[1:2, lo:lo + 8]
    tx_r = meta[2:3, lo:lo + 8]
    ty_r = meta[3:4, lo:lo + 8]
    gw_r = meta[4:5, lo:lo + 8]
    gh_r = meta[5:6, lo:lo + 8]
    cls_r = meta[6:7, lo:lo + 8]
    win_r = meta[7:8, lo:lo + 8]
    uniq_r = meta[8:9, lo:lo + 8]

    row = lax.broadcasted_iota(jnp.int32, (_YX, 8), 0).astype(jnp.float32)
    onehot = jnp.where(row == yx_r, 1.0, 0.0)
    g8 = lax.dot_general(pt, onehot, (((0,), (0,)), ((), ())))
    ga = lax.dot_general(gt_conf, onehot, (((1,), (0,)), ((), ())))

    gsel = jnp.zeros((_CH, 8), jnp.float32)
    a_iota = lax.broadcasted_iota(jnp.int32, (_A, 8), 0).astype(jnp.float32)
    gtc = jnp.sum(jnp.where(a_iota == jb_r, ga, 0.0), axis=0,
                  keepdims=True)
    for a in range(_A):
        gsel = gsel + jnp.where(jb_r == float(a),
                                g8[a * _CH:(a + 1) * _CH], 0.0)

    gxy = jax.nn.sigmoid(gsel[0:2])
    gwh = jnp.exp(gsel[2:4])
    gconf = jax.nn.sigmoid(gsel[4:5])
    gcls = gsel[5:25]

    txty = jnp.concatenate([tx_r, ty_r], axis=0)
    xy_s = jnp.sum(win_r * (txty - gxy) ** 2)

    gtwh = jnp.concatenate([gw_r, gh_r], axis=0)
    wh_s = jnp.sum(win_r * (jnp.sqrt(gtwh) - jnp.sqrt(gwh)) ** 2)

    conf_s = jnp.sum(win_r * (gtc - gconf) ** 2)

    cmax = jnp.max(gcls, axis=0, keepdims=True)
    lse = jnp.log(jnp.sum(jnp.exp(gcls - cmax), axis=0, keepdims=True)) + cmax
    c_iota = lax.broadcasted_iota(jnp.int32, (_C, 8), 0).astype(jnp.float32)
    selc = jnp.sum(jnp.where(c_iota == cls_r, gcls, 0.0), axis=0,
                   keepdims=True)
    cls_s = jnp.sum(uniq_r * (lse - selc))

    s_iota = lax.broadcasted_iota(jnp.int32, (8, 128), 0)
    l_iota = lax.broadcasted_iota(jnp.int32, (8, 128), 1)
    vals = (jnp.where(s_iota == 0, xy_s, 0.0)
            + jnp.where(s_iota == 1, wh_s, 0.0)
            + jnp.where(s_iota == 2, conf_s, 0.0)
            + jnp.where(s_iota == 3, dense_sum, 0.0)
            + jnp.where(s_iota == 4, cls_s, 0.0))
    return jnp.where(l_iota == 0, vals, 0.0)


@jax.jit
def _run(pred_targets, gt_boxes, labf):
    B = pred_targets.shape[0]
    predN = pred_targets.reshape(B, _YX, _A * _CH)
    anc = jnp.asarray(_ANC_PLANES)
    sel = jnp.asarray(_SEL)

    gtc = gt_boxes.reshape(_NOBJ // 16, 16, 4).transpose(0, 2, 1)
    packed = jnp.concatenate(
        [gtc, labf.reshape(_NOBJ // 16, 1, 16)], axis=1).reshape(-1)
    meta = _sc_assign(packed)
    meta3 = meta.reshape(_NOBJ // 16, 16, 16)

    out = pl.pallas_call(
        _tc_kernel,
        grid=(B // _G,),
        in_specs=[
            pl.BlockSpec(memory_space=pltpu.SMEM),
            pl.BlockSpec((_G // 2, 16, 16), lambda b: (b, 0, 0)),
            pl.BlockSpec((_G, _YX, _A * _CH), lambda b: (b, 0, 0)),
            pl.BlockSpec((4 * _A, _YX), lambda b: (0, 0)),
            pl.BlockSpec((_A * _CH, 5 * _A), lambda b: (0, 0)),
        ],
        out_specs=pl.BlockSpec((8, 128), lambda b: (0, 0)),
        out_shape=jax.ShapeDtypeStruct((8, 128), jnp.float32),
    )(gt_boxes, meta3, predN, anc, sel)

    o = out[:, 0]
    l1 = 5.0 * o[0]
    l2 = 5.0 * o[1]
    l3 = o[2]
    l4 = 0.5 * (o[3] - o[2])
    l5 = o[4]
    total = l1 + l2 + l3 + l4 + l5
    return total, (l1, l2, l3, l4, l5)


def kernel(pred_targets, gt_boxes, gt_labels):
    labf = gt_labels.astype(jnp.float32)
    return _run(pred_targets, gt_boxes, labf)


# speedup vs baseline: 1.0159x; 1.0159x over previous
import functools

import jax
import jax.numpy as jnp
import numpy as np
from jax import lax
from jax.experimental import pallas as pl
from jax.experimental.pallas import tpu as pltpu
from jax.experimental.pallas import tpu_sc as plsc

_ANCHORS = np.array(
    [[1.3221, 1.73145], [3.19275, 4.00944], [5.05587, 8.09892],
     [9.47112, 4.84053], [11.2364, 10.0071]], dtype=np.float32)
_A = 5
_C = 20
_S = 13
_YX = _S * _S
_CH = 5 + _C


def _sel_np():
    s = np.zeros((_A * _CH, 5 * _A), dtype=np.float32)
    for f in range(5):
        for a in range(_A):
            s[a * _CH + f, f * _A + a] = 1.0
    return s


def _anc_planes_np():
    ys, xs = np.meshgrid(np.arange(_S, dtype=np.float32),
                         np.arange(_S, dtype=np.float32), indexing='ij')
    cx = (xs + 0.5).reshape(_YX)
    cy = (ys + 0.5).reshape(_YX)
    out = np.zeros((4 * _A, _YX), dtype=np.float32)
    for a in range(_A):
        out[0 * _A + a] = cx
        out[1 * _A + a] = cy
        out[2 * _A + a] = _ANCHORS[a, 0]
        out[3 * _A + a] = _ANCHORS[a, 1]
    return out


_SEL = _sel_np()
_ANC_PLANES = _anc_planes_np()

_G = 32

_NOBJ = 1024
_CHUNK = 16
_NW = 32
_PER_W = _NOBJ // (_CHUNK * _NW)


def _sc_perm(x, idx):
    return lax.gather(
        x, idx.reshape(16, 1),
        lax.GatherDimensionNumbers(offset_dims=(), collapsed_slice_dims=(0,),
                                   start_index_map=(0,)),
        (1,), mode=lax.GatherScatterMode.PROMISE_IN_BOUNDS)


def _sc_assign_kernel(gtflat, labels, meta, in_v, lab_v, out_v):
    cid = lax.axis_index("c")
    sid = lax.axis_index("s")
    wid = sid * 2 + cid
    lane = lax.iota(jnp.int32, 16)
    pos = lane & 7
    quarter = lane >> 2
    for k in range(_PER_W):
        c = wid * _PER_W + k
        pltpu.sync_copy(gtflat.at[pl.ds(c * 64, 64)], in_v)
        pltpu.sync_copy(labels.at[pl.ds(c * 16, 16)], lab_v)
        fields = []
        for f in range(4):
            acc = jnp.zeros((16,), jnp.float32)
            for j in range(4):
                valid = quarter == j
                idx = jnp.where(valid, (lane - 4 * j) * 4 + f, 0)
                vj = _sc_perm(in_v[pl.ds(16 * j, 16)], idx)
                acc = jnp.where(valid, vj, acc)
            fields.append(acc * float(_S))
        x1, y1, x2, y2 = fields
        area_g = (x2 - x1) * (y2 - y1)
        bx = (x1 + x2) * 0.5
        by = (y1 + y2) * 0.5
        bw = x2 - x1
        bh = y2 - y1
        cxf = bx.astype(jnp.int32).astype(jnp.float32)
        cyf = by.astype(jnp.int32).astype(jnp.float32)
        tx = bx - cxf
        ty = by - cyf

        best = jnp.full((16,), -1.0, jnp.float32)
        jb = jnp.zeros((16,), jnp.float32)
        awj = jnp.ones((16,), jnp.float32)
        ahj = jnp.ones((16,), jnp.float32)
        for a in range(_A):
            aw = float(_ANCHORS[a, 0])
            ah = float(_ANCHORS[a, 1])
            iwx = jnp.maximum(
                jnp.minimum(cxf + 0.5 + aw * 0.5, x2)
                - jnp.maximum(cxf + 0.5 - aw * 0.5, x1), 0.0)
            iwy = jnp.maximum(
                jnp.minimum(cyf + 0.5 + ah * 0.5, y2)
                - jnp.maximum(cyf + 0.5 - ah * 0.5, y1), 0.0)
            ai = iwx * iwy
            aiou = ai / (aw * ah + area_g - ai)
            take = aiou > best
            best = jnp.maximum(best, aiou)
            jb = jnp.where(take, float(a), jb)
            awj = jnp.where(take, aw, awj)
            ahj = jnp.where(take, ah, ahj)

        yx = cyf * float(_S) + cxf
        key = yx * float(_A) + jb
        clsf = lab_v[...].astype(jnp.float32) - 1.0

        later_eq = jnp.zeros((16,), jnp.float32)
        later_pair = jnp.zeros((16,), jnp.float32)
        for dd in range(1, 8):
            valid = (pos + dd) < 8
            perm = lane + jnp.where(valid, dd, 0)
            kg = _sc_perm(key, perm)
            cg = _sc_perm(clsf, perm)
            hit = jnp.where(valid & (kg == key), 1.0, 0.0)
            later_eq = jnp.maximum(later_eq, hit)
            later_pair = jnp.maximum(
                later_pair, jnp.where(cg == clsf, hit, 0.0))
        winner = 1.0 - later_eq
        uniq = 1.0 - later_pair

        fields = [yx, jb, tx, ty, bw / awj, bh / ahj, clsf, winner, uniq]
        zero = jnp.zeros((16,), jnp.float32)
        for f in range(16):
            out_v[pl.ds(f * 16, 16)] = fields[f] if f < 9 else zero
        pltpu.sync_copy(out_v, meta.at[pl.ds(c * 256, 256)])


@functools.partial(
    pl.kernel,
    mesh=plsc.VectorSubcoreMesh(core_axis_name="c", subcore_axis_name="s"),
    out_type=jax.ShapeDtypeStruct((_NOBJ * 16,), jnp.float32),
    scratch_types=[
        pltpu.VMEM((64,), jnp.float32),
        pltpu.VMEM((_CHUNK,), jnp.int32),
        pltpu.VMEM((256,), jnp.float32),
    ],
)
def _sc_assign(gtflat, labels, meta, in_v, lab_v, out_v):
    _sc_assign_kernel(gtflat, labels, meta, in_v, lab_v, out_v)


def _tc_kernel(gt_smem, meta_ref, pred_ref, anc_ref, sel_ref, out_ref):
    b = pl.program_id(0)

    @pl.when(b == 0)
    def _init():
        out_ref[...] = jnp.zeros_like(out_ref)

    acc = jnp.zeros((8, 128), jnp.float32)
    for g in range(_G):
        acc = acc + _one_image(pred_ref, g, b * _G + g, gt_smem,
                               meta_ref[g // 2], g % 2,
                               anc_ref[...], sel_ref[...])
    out_ref[...] += acc


def _one_image(pred_ref, g, bb, gt_smem, meta, h, anc, sel):
    pt = pred_ref[g]

    t = lax.dot_general(sel, pt, (((0,), (1,)), ((), ())))
    pxy = jax.nn.sigmoid(t[0:10])
    pwh = jnp.exp(t[10:20])
    pconf = jax.nn.sigmoid(t[20:25])

    cpx = anc[0:5] + pxy[0:5]
    cpy = anc[5:10] + pxy[5:10]
    cpw = anc[10:15] * pwh[0:5]
    cph = anc[15:20] * pwh[5:10]
    px1 = cpx - cpw * 0.5
    py1 = cpy - cph * 0.5
    px2 = cpx + cpw * 0.5
    py2 = cpy + cph * 0.5
    area_p = cpw * cph

    gt_conf = None
    for n in range(8):
        sx1 = gt_smem[bb, n, 0] * float(_S)
        sy1 = gt_smem[bb, n, 1] * float(_S)
        sx2 = gt_smem[bb, n, 2] * float(_S)
        sy2 = gt_smem[bb, n, 3] * float(_S)
        sarea = (sx2 - sx1) * (sy2 - sy1)
        tlx = jnp.maximum(px1, sx1)
        tly = jnp.maximum(py1, sy1)
        brx = jnp.minimum(px2, sx2)
        bry = jnp.minimum(py2, sy2)
        wx = jnp.maximum(brx - tlx, 0.0)
        wy = jnp.maximum(bry - tly, 0.0)
        inter = wx * wy
        iou = inter / (area_p + (sarea - inter))
        gt_conf = iou if gt_conf is None else jnp.maximum(gt_conf, iou)

    d = gt_conf - pconf
    dense_sum = jnp.sum(d * d)

    lo = h * 8
    yx_r = meta[0:1, lo:lo + 8]
    jb_r = meta[1:2, lo:lo + 8]
    tx_r = meta[2:3, lo:lo + 8]
    ty_r = meta[3:4, lo:lo + 8]
    gw_r = meta[4:5, lo:lo + 8]
    gh_r = meta[5:6, lo:lo + 8]
    cls_r = meta[6:7, lo:lo + 8]
    win_r = meta[7:8, lo:lo + 8]
    uniq_r = meta[8:9, lo:lo + 8]

    row = lax.broadcasted_iota(jnp.int32, (_YX, 8), 0).astype(jnp.float32)
    onehot = jnp.where(row == yx_r, 1.0, 0.0)
    g8 = lax.dot_general(pt, onehot, (((0,), (0,)), ((), ())))
    ga = lax.dot_general(gt_conf, onehot, (((1,), (0,)), ((), ())))

    gsel = jnp.zeros((_CH, 8), jnp.float32)
    a_iota = lax.broadcasted_iota(jnp.int32, (_A, 8), 0).astype(jnp.float32)
    gtc = jnp.sum(jnp.where(a_iota == jb_r, ga, 0.0), axis=0,
                  keepdims=True)
    for a in range(_A):
        gsel = gsel + jnp.where(jb_r == float(a),
                                g8[a * _CH:(a + 1) * _CH], 0.0)

    gxy = jax.nn.sigmoid(gsel[0:2])
    gwh = jnp.exp(gsel[2:4])
    gconf = jax.nn.sigmoid(gsel[4:5])
    gcls = gsel[5:25]

    txty = jnp.concatenate([tx_r, ty_r], axis=0)
    xy_s = jnp.sum(win_r * (txty - gxy) ** 2)

    gtwh = jnp.concatenate([gw_r, gh_r], axis=0)
    wh_s = jnp.sum(win_r * (jnp.sqrt(gtwh) - jnp.sqrt(gwh)) ** 2)

    conf_s = jnp.sum(win_r * (gtc - gconf) ** 2)

    cmax = jnp.max(gcls, axis=0, keepdims=True)
    lse = jnp.log(jnp.sum(jnp.exp(gcls - cmax), axis=0, keepdims=True)) + cmax
    c_iota = lax.broadcasted_iota(jnp.int32, (_C, 8), 0).astype(jnp.float32)
    selc = jnp.sum(jnp.where(c_iota == cls_r, gcls, 0.0), axis=0,
                   keepdims=True)
    cls_s = jnp.sum(uniq_r * (lse - selc))

    s_iota = lax.broadcasted_iota(jnp.int32, (8, 128), 0)
    l_iota = lax.broadcasted_iota(jnp.int32, (8, 128), 1)
    vals = (jnp.where(s_iota == 0, xy_s, 0.0)
            + jnp.where(s_iota == 1, wh_s, 0.0)
            + jnp.where(s_iota == 2, conf_s, 0.0)
            + jnp.where(s_iota == 3, dense_sum, 0.0)
            + jnp.where(s_iota == 4, cls_s, 0.0))
    return jnp.where(l_iota == 0, vals, 0.0)


@jax.jit
def _run(pred_targets, gt_boxes, labi):
    B = pred_targets.shape[0]
    predN = pred_targets.reshape(B, _YX, _A * _CH)
    anc = jnp.asarray(_ANC_PLANES)
    sel = jnp.asarray(_SEL)

    meta = _sc_assign(gt_boxes.reshape(-1), labi.reshape(-1))
    meta3 = meta.reshape(_NOBJ // 16, 16, 16)

    out = pl.pallas_call(
        _tc_kernel,
        grid=(B // _G,),
        in_specs=[
            pl.BlockSpec(memory_space=pltpu.SMEM),
            pl.BlockSpec((_G // 2, 16, 16), lambda b: (b, 0, 0)),
            pl.BlockSpec((_G, _YX, _A * _CH), lambda b: (b, 0, 0)),
            pl.BlockSpec((4 * _A, _YX), lambda b: (0, 0)),
            pl.BlockSpec((_A * _CH, 5 * _A), lambda b: (0, 0)),
        ],
        out_specs=pl.BlockSpec((8, 128), lambda b: (0, 0)),
        out_shape=jax.ShapeDtypeStruct((8, 128), jnp.float32),
    )(gt_boxes, meta3, predN, anc, sel)

    o = out[:, 0]
    l1 = 5.0 * o[0]
    l2 = 5.0 * o[1]
    l3 = o[2]
    l4 = 0.5 * (o[3] - o[2])
    l5 = o[4]
    total = l1 + l2 + l3 + l4 + l5
    return total, (l1, l2, l3, l4, l5)


def kernel(pred_targets, gt_boxes, gt_labels):
    return _run(pred_targets, gt_boxes, gt_labels.astype(jnp.int32))
